# baseline (speedup 1.0000x reference)
import functools

import jax
import jax.numpy as jnp
from jax import lax
from jax.experimental import pallas as pl
from jax.experimental.pallas import tpu as pltpu
from jax.experimental.pallas import tpu_sc as plsc

_N = 512
_V = 100000
_L = 16
_CHUNK = 128
_NCHUNK = _N // _CHUNK


def _body(probs_hbm, tgt_hbm, dscr_hbm, mle_hbm,
          out_probs, out_loss,
          tgt_v, fidx_v, rows_v, dscr_v, mle_v, loss_v, sem):
    cid = lax.axis_index("c")
    sid = lax.axis_index("s")

    @pl.when(jnp.logical_and(cid == 0, sid == 0))
    def _():
        pltpu.sync_copy(tgt_hbm, tgt_v)
        pltpu.sync_copy(dscr_hbm, dscr_v)
        pltpu.sync_copy(mle_hbm, mle_v)

        lane = lax.broadcasted_iota(jnp.int32, (_L,), 0)
        acc = jnp.zeros((_L,), jnp.float32)
        for c in range(_NCHUNK):
            for j in range(_CHUNK // _L):
                off = c * _CHUNK + j * _L
                t = tgt_v[pl.ds(off, _L)]
                fidx_v[pl.ds(j * _L, _L)] = t + (lane + off) * _V
            pltpu.async_copy(probs_hbm.at[fidx_v], rows_v, sem).wait()
            pltpu.sync_copy(rows_v, out_probs.at[pl.ds(c * _CHUNK, _CHUNK)])
            for j in range(_CHUNK // _L):
                off = c * _CHUNK + j * _L
                p = rows_v[pl.ds(j * _L, _L)]
                rw = dscr_v[pl.ds(off, _L)] + 0.2 * mle_v[pl.ds(off, _L)]
                acc = acc + p * rw
        for k in (8, 4, 2, 1):
            acc = acc + jnp.take_along_axis(acc, lane ^ k, axis=0)
        loss_v[...] = -acc
        pltpu.sync_copy(loss_v, out_loss)


@jax.jit
def _policy_update(p_flat, targets, dscr, mle):
    call = functools.partial(
        pl.kernel,
        out_type=[
            jax.ShapeDtypeStruct((_N,), jnp.float32),
            jax.ShapeDtypeStruct((_L,), jnp.float32),
        ],
        mesh=plsc.VectorSubcoreMesh(core_axis_name="c", subcore_axis_name="s"),
        scratch_types=[
            pltpu.VMEM((_N,), jnp.int32),
            pltpu.VMEM((_CHUNK,), jnp.int32),
            pltpu.VMEM((_CHUNK,), jnp.float32),
            pltpu.VMEM((_N,), jnp.float32),
            pltpu.VMEM((_N,), jnp.float32),
            pltpu.VMEM((_L,), jnp.float32),
            pltpu.SemaphoreType.DMA,
        ],
    )(_body)
    return call(p_flat, targets, dscr, mle)


def kernel(probs, targets, dscr_rewards, mle_rewards):
    p_flat = probs.reshape((-1,))
    out_probs, out_loss = _policy_update(p_flat, targets, dscr_rewards,
                                         mle_rewards)
    return (out_probs, out_loss[0])


# speedup vs baseline: 1.0031x; 1.0031x over previous
import functools

import jax
import jax.numpy as jnp
from jax import lax
from jax.experimental import pallas as pl
from jax.experimental.pallas import tpu as pltpu
from jax.experimental.pallas import tpu_sc as plsc

_N = 512
_V = 100000
_L = 16
_NW = 16
_BW = _N // _NW


def _body(probs_hbm, tgt_hbm, dscr_hbm, mle_hbm,
          out_probs, out_loss, ex_hbm,
          tgt_v, fidx_v, rows_v, dscr_v, mle_v, contrib_v, all_v, loss_v,
          sem1, sem2, sem3, sem4):
    sid = lax.axis_index("s")
    base = sid * _BW

    c1 = pltpu.async_copy(tgt_hbm.at[pl.ds(base, _BW)], tgt_v, sem1)
    c2 = pltpu.async_copy(dscr_hbm.at[pl.ds(base, _BW)], dscr_v, sem2)
    c3 = pltpu.async_copy(mle_hbm.at[pl.ds(base, _BW)], mle_v, sem3)
    c1.wait()

    lane = lax.broadcasted_iota(jnp.int32, (_L,), 0)
    for j in range(_BW // _L):
        t = tgt_v[pl.ds(j * _L, _L)]
        fidx_v[pl.ds(j * _L, _L)] = t + (lane + base + j * _L) * _V

    pltpu.async_copy(probs_hbm.at[fidx_v], rows_v, sem4).wait()
    cout = pltpu.async_copy(rows_v, out_probs.at[pl.ds(base, _BW)], sem1)

    c2.wait()
    c3.wait()
    acc = jnp.zeros((_L,), jnp.float32)
    for j in range(_BW // _L):
        p = rows_v[pl.ds(j * _L, _L)]
        rw = dscr_v[pl.ds(j * _L, _L)] + 0.2 * mle_v[pl.ds(j * _L, _L)]
        acc = acc + p * rw
    contrib_v[...] = acc
    pltpu.sync_copy(contrib_v, ex_hbm.at[sid])
    cout.wait()
    plsc.subcore_barrier()

    @pl.when(sid == 0)
    def _():
        pltpu.sync_copy(ex_hbm, all_v)
        tot = all_v[0]
        for i in range(1, _NW):
            tot = tot + all_v[i]
        for k in (8, 4, 2, 1):
            tot = tot + jnp.take_along_axis(tot, lane ^ k, axis=0)
        loss_v[...] = -tot
        pltpu.sync_copy(loss_v, out_loss)


@jax.jit
def _policy_update(p_flat, targets, dscr, mle):
    call = functools.partial(
        pl.kernel,
        out_type=[
            jax.ShapeDtypeStruct((_N,), jnp.float32),
            jax.ShapeDtypeStruct((_L,), jnp.float32),
            jax.ShapeDtypeStruct((_NW, _L), jnp.float32),
        ],
        mesh=plsc.VectorSubcoreMesh(
            core_axis_name="c", subcore_axis_name="s", num_cores=1),
        scratch_types=[
            pltpu.VMEM((_BW,), jnp.int32),
            pltpu.VMEM((_BW,), jnp.int32),
            pltpu.VMEM((_BW,), jnp.float32),
            pltpu.VMEM((_BW,), jnp.float32),
            pltpu.VMEM((_BW,), jnp.float32),
            pltpu.VMEM((_L,), jnp.float32),
            pltpu.VMEM((_NW, _L), jnp.float32),
            pltpu.VMEM((_L,), jnp.float32),
            pltpu.SemaphoreType.DMA,
            pltpu.SemaphoreType.DMA,
            pltpu.SemaphoreType.DMA,
            pltpu.SemaphoreType.DMA,
        ],
    )(_body)
    return call(p_flat, targets, dscr, mle)


def kernel(probs, targets, dscr_rewards, mle_rewards):
    p_flat = probs.reshape((-1,))
    out_probs, out_loss, _ = _policy_update(p_flat, targets, dscr_rewards,
                                            mle_rewards)
    return (out_probs, out_loss[0])


# speedup vs baseline: 24.1851x; 24.1110x over previous
import jax
import jax.numpy as jnp
from jax import lax
from jax.experimental import pallas as pl
from jax.experimental.pallas import tpu as pltpu

_N = 512
_V = 100000
_W = 128


def _tc_body(tgt_smem, probs_hbm, tgt2d_ref, dscr_ref, mle_ref,
             out_p, out_l, scratch, sem):
    def issue(i, carry):
        t = tgt_smem[i]
        col = pl.multiple_of((t >> 7) * _W, _W)
        pltpu.make_async_copy(
            probs_hbm.at[pl.ds(i, 1), pl.ds(col, _W)],
            scratch.at[pl.ds(i, 1), :],
            sem,
        ).start()
        return carry

    lax.fori_loop(0, _N, issue, 0, unroll=8)

    def drain(i, carry):
        pltpu.make_async_copy(
            probs_hbm.at[pl.ds(0, 1), pl.ds(0, _W)],
            scratch.at[pl.ds(i, 1), :],
            sem,
        ).wait()
        return carry

    lax.fori_loop(0, _N, drain, 0, unroll=8)

    buf = scratch[...]
    low = tgt2d_ref[...] & (_W - 1)
    lanes = lax.broadcasted_iota(jnp.int32, (_N, _W), 1)
    onehot = (lanes == low).astype(jnp.float32)
    val = jnp.sum(buf * onehot, axis=1, keepdims=True)
    out_p[...] = val
    rw = dscr_ref[...] + 0.2 * mle_ref[...]
    out_l[0, 0] = -jnp.sum(val * rw)


@jax.jit
def _policy_update(p2d, targets, tgt2d, dscr2d, mle2d):
    return pl.pallas_call(
        _tc_body,
        in_specs=[
            pl.BlockSpec(memory_space=pltpu.SMEM),
            pl.BlockSpec(memory_space=pltpu.HBM),
            pl.BlockSpec(memory_space=pltpu.VMEM),
            pl.BlockSpec(memory_space=pltpu.VMEM),
            pl.BlockSpec(memory_space=pltpu.VMEM),
        ],
        out_specs=[
            pl.BlockSpec(memory_space=pltpu.VMEM),
            pl.BlockSpec(memory_space=pltpu.SMEM),
        ],
        out_shape=[
            jax.ShapeDtypeStruct((_N, 1), jnp.float32),
            jax.ShapeDtypeStruct((1, 1), jnp.float32),
        ],
        scratch_shapes=[
            pltpu.VMEM((_N, _W), jnp.float32),
            pltpu.SemaphoreType.DMA,
        ],
    )(targets, p2d, tgt2d, dscr2d, mle2d)


def kernel(probs, targets, dscr_rewards, mle_rewards):
    p2d = probs.reshape((_N, _V))
    out_p, out_l = _policy_update(
        p2d, targets, targets.reshape((_N, 1)),
        dscr_rewards.reshape((_N, 1)), mle_rewards.reshape((_N, 1)))
    return (out_p.reshape((_N,)), out_l[0, 0])


# speedup vs baseline: 24.3451x; 1.0066x over previous
import jax
import jax.numpy as jnp
from jax import lax
from jax.experimental import pallas as pl
from jax.experimental.pallas import tpu as pltpu

_N = 512
_V = 100000
_W = 128


_NQ = 4
_G = _N // _NQ


def _tc_body(tgt_smem, probs_hbm, tgt2d_ref, dscr_ref, mle_ref,
             out_p, out_l, scratch, *sems):
    def issue(i, carry):
        for q in range(_NQ):
            r = i + q * _G
            t = tgt_smem[r]
            col = pl.multiple_of((t >> 7) * _W, _W)
            pltpu.make_async_copy(
                probs_hbm.at[pl.ds(r, 1), pl.ds(col, _W)],
                scratch.at[pl.ds(r, 1), :],
                sems[q],
            ).start()
        return carry

    lax.fori_loop(0, _G, issue, 0, unroll=8)

    for q in range(_NQ):
        pltpu.make_async_copy(
            probs_hbm.at[pl.ds(0, _G), pl.ds(0, _W)],
            scratch.at[pl.ds(q * _G, _G), :],
            sems[q],
        ).wait()

    buf = scratch[...]
    low = tgt2d_ref[...] & (_W - 1)
    val = jnp.take_along_axis(buf, low, axis=1)
    out_p[...] = val
    rw = dscr_ref[...] + 0.2 * mle_ref[...]
    out_l[0, 0] = -jnp.sum(val * rw)


@jax.jit
def _policy_update(p2d, targets, tgt2d, dscr2d, mle2d):
    return pl.pallas_call(
        _tc_body,
        in_specs=[
            pl.BlockSpec(memory_space=pltpu.SMEM),
            pl.BlockSpec(memory_space=pltpu.HBM),
            pl.BlockSpec(memory_space=pltpu.VMEM),
            pl.BlockSpec(memory_space=pltpu.VMEM),
            pl.BlockSpec(memory_space=pltpu.VMEM),
        ],
        out_specs=[
            pl.BlockSpec(memory_space=pltpu.VMEM),
            pl.BlockSpec(memory_space=pltpu.SMEM),
        ],
        out_shape=[
            jax.ShapeDtypeStruct((_N, 1), jnp.float32),
            jax.ShapeDtypeStruct((1, 1), jnp.float32),
        ],
        scratch_shapes=[
            pltpu.VMEM((_N, _W), jnp.float32),
        ] + [pltpu.SemaphoreType.DMA] * _NQ,
    )(targets, p2d, tgt2d, dscr2d, mle2d)


def kernel(probs, targets, dscr_rewards, mle_rewards):
    p2d = probs.reshape((_N, _V))
    out_p, out_l = _policy_update(
        p2d, targets, targets.reshape((_N, 1)),
        dscr_rewards.reshape((_N, 1)), mle_rewards.reshape((_N, 1)))
    return (out_p.reshape((_N,)), out_l[0, 0])


# speedup vs baseline: 24.3499x; 1.0002x over previous
import jax
import jax.numpy as jnp
from jax import lax
from jax.experimental import pallas as pl
from jax.experimental.pallas import tpu as pltpu

_N = 512
_V = 100000
_W = 128

_NQ = 4
_G = _N // _NQ


def _tc_body(tgt_smem, probs_hbm, tgt2d_ref, dscr_ref, mle_ref,
             out_p, out_l, scratch, *sems):
    for q in range(_NQ):
        def issue(i, carry, q=q):
            r = i + q * _G
            t = tgt_smem[r]
            col = pl.multiple_of((t >> 7) * _W, _W)
            pltpu.make_async_copy(
                probs_hbm.at[pl.ds(r, 1), pl.ds(col, _W)],
                scratch.at[pl.ds(r, 1), :],
                sems[q],
            ).start()
            return carry

        lax.fori_loop(0, _G, issue, 0, unroll=8)

    rw = dscr_ref[...] + 0.2 * mle_ref[...]
    low = tgt2d_ref[...] & (_W - 1)
    acc = jnp.zeros((), jnp.float32)
    for q in range(_NQ):
        pltpu.make_async_copy(
            probs_hbm.at[pl.ds(0, _G), pl.ds(0, _W)],
            scratch.at[pl.ds(q * _G, _G), :],
            sems[q],
        ).wait()
        sl = pl.ds(q * _G, _G)
        lo, hi = q * _G, (q + 1) * _G
        val = jnp.take_along_axis(scratch[sl, :], low[lo:hi, :], axis=1)
        out_p[sl, :] = val
        acc = acc + jnp.sum(val * rw[lo:hi, :])
    out_l[0, 0] = -acc


@jax.jit
def _policy_update(p2d, targets, tgt2d, dscr2d, mle2d):
    return pl.pallas_call(
        _tc_body,
        in_specs=[
            pl.BlockSpec(memory_space=pltpu.SMEM),
            pl.BlockSpec(memory_space=pltpu.HBM),
            pl.BlockSpec(memory_space=pltpu.VMEM),
            pl.BlockSpec(memory_space=pltpu.VMEM),
            pl.BlockSpec(memory_space=pltpu.VMEM),
        ],
        out_specs=[
            pl.BlockSpec(memory_space=pltpu.VMEM),
            pl.BlockSpec(memory_space=pltpu.SMEM),
        ],
        out_shape=[
            jax.ShapeDtypeStruct((_N, 1), jnp.float32),
            jax.ShapeDtypeStruct((1, 1), jnp.float32),
        ],
        scratch_shapes=[
            pltpu.VMEM((_N, _W), jnp.float32),
        ] + [pltpu.SemaphoreType.DMA] * _NQ,
    )(targets, p2d, tgt2d, dscr2d, mle2d)


def kernel(probs, targets, dscr_rewards, mle_rewards):
    p2d = probs.reshape((_N, _V))
    out_p, out_l = _policy_update(
        p2d, targets, targets.reshape((_N, 1)),
        dscr_rewards.reshape((_N, 1)), mle_rewards.reshape((_N, 1)))
    return (out_p.reshape((_N,)), out_l[0, 0])
